# baseline (speedup 1.0000x reference)
import functools
import math

import jax
import jax.numpy as jnp
from jax.experimental import pallas as pl
from jax.experimental.pallas import tpu as pltpu

_TOP_K = 2
_CAP_FACTOR = 1.25


def _silu(v):
    return v * jax.nn.sigmoid(v)


def _shared_kernel(x_ref, sg_ref, su_ref, sd_ref, r_ref, o_ref):
    x = x_ref[...].astype(jnp.bfloat16)
    sg = sg_ref[...].astype(jnp.bfloat16)
    su = su_ref[...].astype(jnp.bfloat16)
    sd = sd_ref[...].astype(jnp.bfloat16)
    g = jnp.dot(x, sg, preferred_element_type=jnp.float32)
    u = jnp.dot(x, su, preferred_element_type=jnp.float32)
    h = (_silu(g) * u).astype(jnp.bfloat16)
    o_ref[...] = r_ref[...] + jnp.dot(h, sd,
                                      preferred_element_type=jnp.float32)


def _routing_kernel(x_ref, wr_ref, slt_ref, wt_ref, *, cap, n_chunk):
    x = x_ref[...]
    wr = wr_ref[...]
    n, _ = x.shape
    e_dim = wr.shape[1]

    logits = jnp.dot(x, wr, preferred_element_type=jnp.float32)
    m = jnp.max(logits, axis=-1, keepdims=True)
    p = jnp.exp(logits - m)
    p = p / jnp.sum(p, axis=-1, keepdims=True)

    e_iota = jax.lax.broadcasted_iota(jnp.int32, p.shape, 1)
    i1 = jnp.argmax(p, axis=-1, keepdims=True)
    oh1 = e_iota == i1
    p2 = jnp.where(oh1, -1.0, p)
    i2 = jnp.argmax(p2, axis=-1, keepdims=True)
    oh2 = e_iota == i2
    w1 = jnp.max(p, axis=-1, keepdims=True)
    w2 = jnp.max(p2, axis=-1, keepdims=True)
    denom = jnp.maximum(w1 + w2, 1e-9)
    w_mat = jnp.where(oh1, w1 / denom, 0.0) + jnp.where(oh2, w2 / denom, 0.0)
    a_mat = (oh1 | oh2).astype(jnp.float32)

    loads = jnp.max(jnp.sum(a_mat, axis=0, keepdims=True))
    overflow = loads > float(cap)

    @pl.when(jnp.logical_not(overflow))
    def _():
        r_iota = jax.lax.broadcasted_iota(jnp.int32, (n, n), 0)
        c_iota = jax.lax.broadcasted_iota(jnp.int32, (n, n), 1)
        lt = (c_iota < r_iota).astype(jnp.bfloat16)
        slot = jnp.dot(lt, a_mat.astype(jnp.bfloat16),
                       preferred_element_type=jnp.float32)
        slot_mat = jnp.where(a_mat > 0.0, slot, -1.0)
        slt_ref[...] = slot_mat.T
        wt_ref[...] = w_mat.T

    @pl.when(overflow)
    def _():
        w_t = w_mat.T
        a_t = a_mat.T
        j_row = jax.lax.broadcasted_iota(jnp.int32, (1, n), 1).astype(
            jnp.float32)
        slot_cols = []
        for e in range(e_dim):
            w_row = w_t[e:e + 1, :]
            a_row = a_t[e:e + 1, :]
            chunks = []
            for c0 in range(0, n, n_chunk):
                w_col = w_mat[c0:c0 + n_chunk, e:e + 1]
                a_col = a_mat[c0:c0 + n_chunk, e:e + 1]
                i_col = (jax.lax.broadcasted_iota(
                    jnp.int32, (n_chunk, 1), 0).astype(jnp.float32)
                    + float(c0))
                beats = (w_row > w_col) | ((w_row == w_col) & (j_row < i_col))
                rank = jnp.sum(a_row * beats.astype(jnp.float32), axis=1,
                               keepdims=True)
                keep = (a_col > 0.0) & (rank < float(cap))
                chunks.append(jnp.where(keep, rank, -1.0))
            slot_cols.append(jnp.concatenate(chunks, axis=0))
        slot_mat = jnp.concatenate(slot_cols, axis=1)

        kept = slot_mat >= 0.0
        slt_ref[...] = slot_mat.T
        wt_ref[...] = jnp.where(kept, w_mat, 0.0).T


def _expert_kernel(x_ref, slt_ref, wt_ref, wg_ref, wu_ref, wd_ref,
                   o_ref, xg_ref, oacc_ref, p_ref, *, cap):
    e = pl.program_id(0)
    k = pl.program_id(1)
    nk = pl.num_programs(1)
    n = x_ref.shape[0]

    @pl.when(k == 0)
    def _():
        sl_e = slt_ref[pl.ds(e, 1), :]
        c_iota = jax.lax.broadcasted_iota(jnp.int32, (cap, n), 0).astype(
            jnp.float32)
        p_ref[...] = (c_iota == sl_e).astype(jnp.bfloat16)
        xg_ref[...] = jnp.dot(
            p_ref[...], x_ref[...].astype(jnp.bfloat16),
            preferred_element_type=jnp.float32).astype(jnp.bfloat16)

    xg = xg_ref[...]
    g = jnp.dot(xg, wg_ref[0].astype(jnp.bfloat16),
                preferred_element_type=jnp.float32)
    u = jnp.dot(xg, wu_ref[0].astype(jnp.bfloat16),
                preferred_element_type=jnp.float32)
    part = jnp.dot((_silu(g) * u).astype(jnp.bfloat16),
                   wd_ref[0].astype(jnp.bfloat16),
                   preferred_element_type=jnp.float32)

    @pl.when(k == 0)
    def _():
        oacc_ref[...] = part

    @pl.when(k != 0)
    def _():
        oacc_ref[...] += part

    @pl.when(k == nk - 1)
    def _():
        w_e = wt_ref[pl.ds(e, 1), :].astype(jnp.bfloat16)
        p_w = p_ref[...] * w_e
        contrib = jax.lax.dot_general(
            p_w, oacc_ref[...].astype(jnp.bfloat16), (((0,), (0,)), ((), ())),
            preferred_element_type=jnp.float32)

        @pl.when(e == 0)
        def _():
            o_ref[...] = contrib

        @pl.when(e != 0)
        def _():
            o_ref[...] += contrib


def kernel(x, Wr, Wg, Wu, Wd, Sg, Su, Sd):
    b, s, h = x.shape
    n = b * s
    e_dim = Wr.shape[1]
    inner = Wg.shape[2]
    cap = max(1, math.ceil(n * _TOP_K / e_dim * _CAP_FACTOR))

    flat_x = x.reshape(n, h)

    slt, wt = pl.pallas_call(
        functools.partial(_routing_kernel, cap=cap, n_chunk=512),
        in_specs=[
            pl.BlockSpec((n, h), lambda: (0, 0)),
            pl.BlockSpec((h, e_dim), lambda: (0, 0)),
        ],
        out_specs=[
            pl.BlockSpec((e_dim, n), lambda: (0, 0)),
            pl.BlockSpec((e_dim, n), lambda: (0, 0)),
        ],
        out_shape=[
            jax.ShapeDtypeStruct((e_dim, n), jnp.float32),
            jax.ShapeDtypeStruct((e_dim, n), jnp.float32),
        ],
    )(flat_x, Wr)

    nk = 4
    ib = inner // nk
    routed = pl.pallas_call(
        functools.partial(_expert_kernel, cap=cap),
        grid=(e_dim, nk),
        in_specs=[
            pl.BlockSpec((n, h), lambda e, k: (0, 0)),
            pl.BlockSpec((e_dim, n), lambda e, k: (0, 0)),
            pl.BlockSpec((e_dim, n), lambda e, k: (0, 0)),
            pl.BlockSpec((1, h, ib), lambda e, k: (e, 0, k)),
            pl.BlockSpec((1, h, ib), lambda e, k: (e, 0, k)),
            pl.BlockSpec((1, ib, h), lambda e, k: (e, k, 0)),
        ],
        out_specs=pl.BlockSpec((n, h), lambda e, k: (0, 0)),
        out_shape=jax.ShapeDtypeStruct((n, h), jnp.float32),
        scratch_shapes=[
            pltpu.VMEM((cap, h), jnp.bfloat16),
            pltpu.VMEM((cap, h), jnp.float32),
            pltpu.VMEM((cap, n), jnp.bfloat16),
        ],
    )(flat_x, slt, wt, Wg, Wu, Wd)

    n_blk = 4
    out = pl.pallas_call(
        _shared_kernel,
        grid=(n_blk,),
        in_specs=[
            pl.BlockSpec((n // n_blk, h), lambda i: (i, 0)),
            pl.BlockSpec((h, inner), lambda i: (0, 0)),
            pl.BlockSpec((h, inner), lambda i: (0, 0)),
            pl.BlockSpec((inner, h), lambda i: (0, 0)),
            pl.BlockSpec((n // n_blk, h), lambda i: (i, 0)),
        ],
        out_specs=pl.BlockSpec((n // n_blk, h), lambda i: (i, 0)),
        out_shape=jax.ShapeDtypeStruct((n, h), jnp.float32),
    )(flat_x, Sg, Su, Sd, routed)

    return out.reshape(b, s, h)


# speedup vs baseline: 1.0214x; 1.0214x over previous
import functools
import math

import jax
import jax.numpy as jnp
from jax.experimental import pallas as pl
from jax.experimental.pallas import tpu as pltpu

_TOP_K = 2
_CAP_FACTOR = 1.25


def _silu(v):
    return v * jax.nn.sigmoid(v)


def _shared_kernel(x_ref, sg_ref, su_ref, sd_ref, r_ref, o_ref):
    x = x_ref[...].astype(jnp.bfloat16)
    sg = sg_ref[...].astype(jnp.bfloat16)
    su = su_ref[...].astype(jnp.bfloat16)
    sd = sd_ref[...].astype(jnp.bfloat16)
    g = jnp.dot(x, sg, preferred_element_type=jnp.float32)
    u = jnp.dot(x, su, preferred_element_type=jnp.float32)
    h = (_silu(g) * u).astype(jnp.bfloat16)
    o_ref[...] = r_ref[...] + jnp.dot(h, sd,
                                      preferred_element_type=jnp.float32)


def _routing_kernel(x_ref, wr_ref, slt_ref, wt_ref, *, cap, n_chunk):
    x = x_ref[...]
    wr = wr_ref[...]
    n, _ = x.shape
    e_dim = wr.shape[1]

    logits = jnp.dot(x, wr, preferred_element_type=jnp.float32)
    m = jnp.max(logits, axis=-1, keepdims=True)
    p = jnp.exp(logits - m)
    p = p / jnp.sum(p, axis=-1, keepdims=True)

    e_iota = jax.lax.broadcasted_iota(jnp.int32, p.shape, 1)
    i1 = jnp.argmax(p, axis=-1, keepdims=True)
    oh1 = e_iota == i1
    p2 = jnp.where(oh1, -1.0, p)
    i2 = jnp.argmax(p2, axis=-1, keepdims=True)
    oh2 = e_iota == i2
    w1 = jnp.max(p, axis=-1, keepdims=True)
    w2 = jnp.max(p2, axis=-1, keepdims=True)
    denom = jnp.maximum(w1 + w2, 1e-9)
    w_mat = jnp.where(oh1, w1 / denom, 0.0) + jnp.where(oh2, w2 / denom, 0.0)
    a_mat = (oh1 | oh2).astype(jnp.float32)

    loads = jnp.max(jnp.sum(a_mat, axis=0, keepdims=True))
    overflow = loads > float(cap)

    @pl.when(jnp.logical_not(overflow))
    def _():
        r_iota = jax.lax.broadcasted_iota(jnp.int32, (n, n), 0)
        c_iota = jax.lax.broadcasted_iota(jnp.int32, (n, n), 1)
        lt = (c_iota < r_iota).astype(jnp.bfloat16)
        slot = jnp.dot(lt, a_mat.astype(jnp.bfloat16),
                       preferred_element_type=jnp.float32)
        slot_mat = jnp.where(a_mat > 0.0, slot, -1.0)
        slt_ref[...] = slot_mat.T
        wt_ref[...] = w_mat.T

    @pl.when(overflow)
    def _():
        w_t = w_mat.T
        a_t = a_mat.T
        j_row = jax.lax.broadcasted_iota(jnp.int32, (1, n), 1).astype(
            jnp.float32)
        slot_cols = []
        for e in range(e_dim):
            w_row = w_t[e:e + 1, :]
            a_row = a_t[e:e + 1, :]
            chunks = []
            for c0 in range(0, n, n_chunk):
                w_col = w_mat[c0:c0 + n_chunk, e:e + 1]
                a_col = a_mat[c0:c0 + n_chunk, e:e + 1]
                i_col = (jax.lax.broadcasted_iota(
                    jnp.int32, (n_chunk, 1), 0).astype(jnp.float32)
                    + float(c0))
                beats = (w_row > w_col) | ((w_row == w_col) & (j_row < i_col))
                rank = jnp.sum(a_row * beats.astype(jnp.float32), axis=1,
                               keepdims=True)
                keep = (a_col > 0.0) & (rank < float(cap))
                chunks.append(jnp.where(keep, rank, -1.0))
            slot_cols.append(jnp.concatenate(chunks, axis=0))
        slot_mat = jnp.concatenate(slot_cols, axis=1)

        kept = slot_mat >= 0.0
        slt_ref[...] = slot_mat.T
        wt_ref[...] = jnp.where(kept, w_mat, 0.0).T


def _expert_kernel(x_ref, slt_ref, wt_ref, wg_ref, wu_ref, wd_ref,
                   o_ref, xg_ref, oacc_ref, *, cap):
    e = pl.program_id(0)
    k = pl.program_id(1)
    nk = pl.num_programs(1)
    n = x_ref.shape[0]

    def one_hot():
        sl_e = slt_ref[pl.ds(e, 1), :]
        c_iota = jax.lax.broadcasted_iota(jnp.int32, (cap, n), 0).astype(
            jnp.float32)
        return (c_iota == sl_e).astype(jnp.bfloat16)

    @pl.when(k == 0)
    def _():
        xg_ref[...] = jnp.dot(
            one_hot(), x_ref[...].astype(jnp.bfloat16),
            preferred_element_type=jnp.float32).astype(jnp.bfloat16)

    xg = xg_ref[...]
    g = jnp.dot(xg, wg_ref[0].astype(jnp.bfloat16),
                preferred_element_type=jnp.float32)
    u = jnp.dot(xg, wu_ref[0].astype(jnp.bfloat16),
                preferred_element_type=jnp.float32)
    part = jnp.dot((_silu(g) * u).astype(jnp.bfloat16),
                   wd_ref[0].astype(jnp.bfloat16),
                   preferred_element_type=jnp.float32)

    @pl.when(k == 0)
    def _():
        oacc_ref[...] = part

    @pl.when(k != 0)
    def _():
        oacc_ref[...] += part

    @pl.when(k == nk - 1)
    def _():
        w_e = wt_ref[pl.ds(e, 1), :].astype(jnp.bfloat16)
        p_w = one_hot() * w_e
        contrib = jax.lax.dot_general(
            p_w, oacc_ref[...].astype(jnp.bfloat16), (((0,), (0,)), ((), ())),
            preferred_element_type=jnp.float32)

        @pl.when(e == 0)
        def _():
            o_ref[...] = contrib

        @pl.when(e != 0)
        def _():
            o_ref[...] += contrib


def kernel(x, Wr, Wg, Wu, Wd, Sg, Su, Sd):
    b, s, h = x.shape
    n = b * s
    e_dim = Wr.shape[1]
    inner = Wg.shape[2]
    cap = max(1, math.ceil(n * _TOP_K / e_dim * _CAP_FACTOR))

    flat_x = x.reshape(n, h)

    slt, wt = pl.pallas_call(
        functools.partial(_routing_kernel, cap=cap, n_chunk=512),
        in_specs=[
            pl.BlockSpec((n, h), lambda: (0, 0)),
            pl.BlockSpec((h, e_dim), lambda: (0, 0)),
        ],
        out_specs=[
            pl.BlockSpec((e_dim, n), lambda: (0, 0)),
            pl.BlockSpec((e_dim, n), lambda: (0, 0)),
        ],
        out_shape=[
            jax.ShapeDtypeStruct((e_dim, n), jnp.float32),
            jax.ShapeDtypeStruct((e_dim, n), jnp.float32),
        ],
    )(flat_x, Wr)

    nk = 2
    ib = inner // nk
    routed = pl.pallas_call(
        functools.partial(_expert_kernel, cap=cap),
        grid=(e_dim, nk),
        in_specs=[
            pl.BlockSpec((n, h), lambda e, k: (0, 0)),
            pl.BlockSpec((e_dim, n), lambda e, k: (0, 0)),
            pl.BlockSpec((e_dim, n), lambda e, k: (0, 0)),
            pl.BlockSpec((1, h, ib), lambda e, k: (e, 0, k)),
            pl.BlockSpec((1, h, ib), lambda e, k: (e, 0, k)),
            pl.BlockSpec((1, ib, h), lambda e, k: (e, k, 0)),
        ],
        out_specs=pl.BlockSpec((n, h), lambda e, k: (0, 0)),
        out_shape=jax.ShapeDtypeStruct((n, h), jnp.float32),
        scratch_shapes=[
            pltpu.VMEM((cap, h), jnp.bfloat16),
            pltpu.VMEM((cap, h), jnp.float32),
        ],
    )(flat_x, slt, wt, Wg, Wu, Wd)

    n_blk = 4
    out = pl.pallas_call(
        _shared_kernel,
        grid=(n_blk,),
        in_specs=[
            pl.BlockSpec((n // n_blk, h), lambda i: (i, 0)),
            pl.BlockSpec((h, inner), lambda i: (0, 0)),
            pl.BlockSpec((h, inner), lambda i: (0, 0)),
            pl.BlockSpec((inner, h), lambda i: (0, 0)),
            pl.BlockSpec((n // n_blk, h), lambda i: (i, 0)),
        ],
        out_specs=pl.BlockSpec((n // n_blk, h), lambda i: (i, 0)),
        out_shape=jax.ShapeDtypeStruct((n, h), jnp.float32),
    )(flat_x, Sg, Su, Sd, routed)

    return out.reshape(b, s, h)
